# baseline (speedup 1.0000x reference)
import functools

import jax
import jax.numpy as jnp
from jax import lax
from jax.experimental import pallas as pl
from jax.experimental.pallas import tpu as pltpu
from jax.experimental.pallas import tpu_sc as plsc

N = 50000
E = 800000
H = 64

_info = plsc.get_sparse_core_info()
NC, NS = _info.num_cores, _info.num_subcores
NW = NC * NS
NB = NW
NPB = 1568
LASTROWS = N - (NB - 1) * NPB
EPW = E // NW
RW = 26112
E_PAD = NW * RW
E_PAD2 = E_PAD + 2048
BCH = 1000
GC = 256
SEG = 32


def _wid():
    return lax.axis_index("s") * NC + lax.axis_index("c")


def _bucket_of(d):
    return ((d >> 5) * 669) >> 15


_VECS = BCH // 16
_REM = BCH - _VECS * 16


def _lane_extract(vec_a, vec_b, b):
    return vec_a[b] if b < 16 else vec_b[b - 16]


def _bucketize_body(
    dst_hbm,
    src_hbm,
    dstg_hbm,
    srcg_hbm,
    perm_hbm,
    dloc_hbm,
    pref_hbm,
    pcnt_hbm,
    chd,
    chs,
    reg_d,
    reg_s,
    reg_p,
    reg_l,
    prefbuf,
    pcntbuf,
):
    w = _wid()
    wbase = w * EPW
    obase = w * RW
    iota = lax.iota(jnp.int32, 16)
    zero16 = jnp.zeros((16,), jnp.int32)

    def zr(k, carry):
        reg_d[pl.ds(k * 16, 16)] = zero16
        reg_s[pl.ds(k * 16, 16)] = zero16
        reg_p[pl.ds(k * 16, 16)] = zero16
        reg_l[pl.ds(k * 16, 16)] = zero16
        return carry

    lax.fori_loop(0, RW // 16, zr, 0)

    def pass_a(i, acc):
        acc_a, acc_b = acc
        pltpu.sync_copy(dst_hbm.at[pl.ds(wbase + i * BCH, BCH)], chd.at[pl.ds(0, BCH)])

        def inner(v, acc2):
            a2a, a2b = acc2
            d16 = chd[pl.ds(v * 16, 16)]
            nvalid = jnp.where(v == _VECS, _REM, 16)
            valid = iota < nvalid
            b16 = _bucket_of(d16)
            for b in range(NB):
                mk = jnp.logical_and(b16 == b, valid)
                pop = plsc.all_reduce_population_count(mk)
                oh = (iota == (b % 16)).astype(jnp.int32) * pop
                if b < 16:
                    a2a = a2a + oh
                else:
                    a2b = a2b + oh
            return (a2a, a2b)

        return lax.fori_loop(0, _VECS + 1, inner, (acc_a, acc_b))

    cnt_a, cnt_b = lax.fori_loop(0, EPW // BCH, pass_a, (zero16, zero16))

    pc_a = (cnt_a + (SEG - 1)) & (-SEG)
    pc_b = (cnt_b + (SEG - 1)) & (-SEG)
    inc_a = jnp.cumsum(pc_a)
    inc_b = jnp.cumsum(pc_b)
    ex_a = inc_a - pc_a
    ex_b = inc_b - pc_b + inc_a[15]

    prefbuf[pl.ds(0, 16)] = ex_a
    prefbuf[pl.ds(16, 16)] = ex_b
    pcntbuf[pl.ds(0, 16)] = pc_a
    pcntbuf[pl.ds(16, 16)] = pc_b
    pltpu.sync_copy(prefbuf, pref_hbm.at[pl.ds(w * NB, NB)])
    pltpu.sync_copy(pcntbuf, pcnt_hbm.at[pl.ds(w * NB, NB)])

    def pass_b(i, cur):
        cur_a, cur_b = cur
        cbase = wbase + i * BCH
        pltpu.sync_copy(dst_hbm.at[pl.ds(cbase, BCH)], chd.at[pl.ds(0, BCH)])
        pltpu.sync_copy(src_hbm.at[pl.ds(cbase, BCH)], chs.at[pl.ds(0, BCH)])

        def inner(v, cur2):
            c2a, c2b = cur2
            d16 = chd[pl.ds(v * 16, 16)]
            s16 = chs[pl.ds(v * 16, 16)]
            nvalid = jnp.where(v == _VECS, _REM, 16)
            valid = iota < nvalid
            b16 = _bucket_of(d16)
            pos = zero16
            for b in range(NB):
                mk = jnp.logical_and(b16 == b, valid)
                pc = jnp.cumsum(mk.astype(jnp.int32))
                pop = pc[15]
                cb = _lane_extract(c2a, c2b, b)
                pos = jnp.where(mk, cb + pc - 1, pos)
                oh = (iota == (b % 16)).astype(jnp.int32) * pop
                if b < 16:
                    c2a = c2a + oh
                else:
                    c2b = c2b + oh
            gid16 = cbase + v * 16 + iota
            dl16 = d16 - b16 * NPB
            plsc.store_scatter(reg_d, [pos], d16, mask=valid)
            plsc.store_scatter(reg_s, [pos], s16, mask=valid)
            plsc.store_scatter(reg_p, [pos], gid16, mask=valid)
            plsc.store_scatter(reg_l, [pos], dl16, mask=valid)
            return (c2a, c2b)

        return lax.fori_loop(0, _VECS + 1, inner, (cur_a, cur_b))

    lax.fori_loop(0, EPW // BCH, pass_b, (ex_a, ex_b))

    for b in range(NB):
        c = _lane_extract(cnt_a, cnt_b, b)
        pc = _lane_extract(pc_a, pc_b, b)
        start = _lane_extract(ex_a, ex_b, b)
        pad = pc - c
        seg_end = start + c

        @pl.when(pad > 0)
        def _():
            lastidx = jnp.full((16,), seg_end - 1, jnp.int32)
            lv_d = plsc.load_gather(reg_d, [lastidx])
            lv_s = plsc.load_gather(reg_s, [lastidx])
            lv_p = plsc.load_gather(reg_p, [lastidx])
            lv_l = plsc.load_gather(reg_l, [lastidx])
            for j in (0, 16):
                lanes = iota + j
                mask = lanes < pad
                idx = seg_end + lanes
                plsc.store_scatter(reg_d, [idx], lv_d, mask=mask)
                plsc.store_scatter(reg_s, [idx], lv_s, mask=mask)
                plsc.store_scatter(reg_p, [idx], lv_p, mask=mask)
                plsc.store_scatter(reg_l, [idx], lv_l, mask=mask)

    pltpu.sync_copy(reg_d, dstg_hbm.at[pl.ds(obase, RW)])
    pltpu.sync_copy(reg_s, srcg_hbm.at[pl.ds(obase, RW)])
    pltpu.sync_copy(reg_p, perm_hbm.at[pl.ds(obase, RW)])
    pltpu.sync_copy(reg_l, dloc_hbm.at[pl.ds(obase, RW)])


_sc_bucketize = functools.partial(
    pl.kernel,
    mesh=plsc.VectorSubcoreMesh(core_axis_name="c", subcore_axis_name="s"),
    out_type=[
        jax.ShapeDtypeStruct((E_PAD,), jnp.int32),
        jax.ShapeDtypeStruct((E_PAD,), jnp.int32),
        jax.ShapeDtypeStruct((E_PAD,), jnp.int32),
        jax.ShapeDtypeStruct((E_PAD2,), jnp.int32),
        jax.ShapeDtypeStruct((NW * NB,), jnp.int32),
        jax.ShapeDtypeStruct((NW * NB,), jnp.int32),
    ],
    scratch_types=[
        pltpu.VMEM((BCH + 16,), jnp.int32),
        pltpu.VMEM((BCH + 16,), jnp.int32),
        pltpu.VMEM((RW,), jnp.int32),
        pltpu.VMEM((RW,), jnp.int32),
        pltpu.VMEM((RW,), jnp.int32),
        pltpu.VMEM((RW,), jnp.int32),
        pltpu.VMEM((NB,), jnp.int32),
        pltpu.VMEM((NB,), jnp.int32),
    ],
    compiler_params=pltpu.CompilerParams(needs_layout_passes=False),
)(_bucketize_body)


def _permute_body(perm_hbm, ea2_hbm, eap_hbm, idx0, idx1, row0, row1, si0, si1, sg0, sg1, sw0, sw1):
    w = _wid()
    obase = w * RW
    nch = RW // GC
    idxb = (idx0, idx1)
    rowb = (row0, row1)
    sib = (si0, si1)
    sgb = (sg0, sg1)
    swb = (sw0, sw1)

    for b in (0, 1):
        pltpu.async_copy(perm_hbm.at[pl.ds(obase + b * GC, GC)], idxb[b], sib[b])
    pltpu.make_async_copy(perm_hbm.at[pl.ds(obase, GC)], idxb[0], sib[0]).wait()
    pltpu.async_copy(ea2_hbm.at[idxb[0]], rowb[0], sgb[0])

    def step(g, carry):
        for b in (1, 0):
            i = 2 * g + (1 if b == 1 else 2)
            pb = 1 - b

            @pl.when(i < nch)
            def _():
                base = obase + i * GC

                @pl.when(i >= 2)
                def _():
                    pltpu.make_async_copy(rowb[b], eap_hbm.at[pl.ds(base - 2 * GC, GC)], swb[b]).wait()

                pltpu.make_async_copy(perm_hbm.at[pl.ds(base, GC)], idxb[b], sib[b]).wait()
                pltpu.async_copy(ea2_hbm.at[idxb[b]], rowb[b], sgb[b])
                pltpu.make_async_copy(ea2_hbm.at[idxb[pb]], rowb[pb], sgb[pb]).wait()

                @pl.when(i + 1 < nch)
                def _():
                    pltpu.async_copy(perm_hbm.at[pl.ds(base + GC, GC)], idxb[pb], sib[pb])

                pltpu.async_copy(rowb[pb], eap_hbm.at[pl.ds(base - GC, GC)], swb[pb])
        return carry

    lax.fori_loop(0, nch // 2, step, 0)
    lastb = (nch - 1) & 1
    pltpu.make_async_copy(ea2_hbm.at[idxb[lastb]], rowb[lastb], sgb[lastb]).wait()
    pltpu.async_copy(rowb[lastb], eap_hbm.at[pl.ds(obase + (nch - 1) * GC, GC)], swb[lastb])
    for b in (0, 1):
        base = obase + (nch - 2 + b) * GC
        pltpu.make_async_copy(rowb[b], eap_hbm.at[pl.ds(base, GC)], swb[b]).wait()


_sc_permute = functools.partial(
    pl.kernel,
    mesh=plsc.VectorSubcoreMesh(core_axis_name="c", subcore_axis_name="s"),
    out_type=jax.ShapeDtypeStruct((E_PAD2, 2 * H), jnp.float32),
    scratch_types=[
        pltpu.VMEM((GC,), jnp.int32),
        pltpu.VMEM((GC,), jnp.int32),
        pltpu.VMEM((GC, 2 * H), jnp.float32),
        pltpu.VMEM((GC, 2 * H), jnp.float32),
        pltpu.SemaphoreType.DMA,
        pltpu.SemaphoreType.DMA,
        pltpu.SemaphoreType.DMA,
        pltpu.SemaphoreType.DMA,
        pltpu.SemaphoreType.DMA,
        pltpu.SemaphoreType.DMA,
    ],
)(_permute_body)


GGC = 256

def _gather_body(dstg_hbm, srcg_hbm, t1_hbm, t2_hbm, g1_hbm, g2_hbm, idx_v, rows_v, sem):
    w = _wid()
    obase = w * RW
    nch = RW // GGC

    def chunk(i, carry):
        base = obase + i * GGC
        pltpu.sync_copy(dstg_hbm.at[pl.ds(base, GGC)], idx_v)
        pltpu.async_copy(t1_hbm.at[idx_v], rows_v, sem).wait()
        pltpu.sync_copy(rows_v, g1_hbm.at[pl.ds(base, GGC)])
        pltpu.sync_copy(srcg_hbm.at[pl.ds(base, GGC)], idx_v)
        pltpu.async_copy(t2_hbm.at[idx_v], rows_v, sem).wait()
        pltpu.sync_copy(rows_v, g2_hbm.at[pl.ds(base, GGC)])
        return carry

    lax.fori_loop(0, nch, chunk, 0)


_sc_gather = functools.partial(
    pl.kernel,
    mesh=plsc.VectorSubcoreMesh(core_axis_name="c", subcore_axis_name="s"),
    out_type=[
        jax.ShapeDtypeStruct((E_PAD2, 2 * H), jnp.float32),
        jax.ShapeDtypeStruct((E_PAD2, 2 * H), jnp.float32),
    ],
    scratch_types=[
        pltpu.VMEM((GGC,), jnp.int32),
        pltpu.VMEM((GGC, 2 * H), jnp.float32),
        pltpu.SemaphoreType.DMA,
    ],
)(_gather_body)


def _sgeti(ref, i):
    return ref[pl.ds(i, 16)][0]


SB = 64


def _scatter_body(
    m_hbm, dloc_hbm, pref_hbm, pcnt_hbm, agg_hbm,
    table, mb0, mb1, dl0, dl1, prefv, pcntv, s0, s1,
):
    b = _wid()
    pltpu.sync_copy(pref_hbm, prefv.at[pl.ds(0, NW * NB)])
    pltpu.sync_copy(pcnt_hbm, pcntv.at[pl.ds(0, NW * NB)])
    mbb = (mb0, mb1)
    dlb = (dl0, dl1)
    smb = (s0, s1)

    zero16 = jnp.zeros((16,), jnp.float32)

    def zr(r, carry):
        table[pl.ds(r * 16, 16)] = zero16
        return carry

    lax.fori_loop(0, NPB * H // 16, zr, 0)

    def per_worker(u, carry):
        base = pl.multiple_of(u * RW + _sgeti(prefv, u * NB + b), SEG)
        seg_len = _sgeti(pcntv, u * NB + b)
        nblk = (seg_len + SB - 1) >> 6

        def issue(k, sl):
            off = pl.multiple_of(base + k * SB, SEG)
            pltpu.async_copy(dloc_hbm.at[pl.ds(off, SB)], dlb[sl].at[pl.ds(0, SB)], smb[sl])
            pltpu.async_copy(m_hbm.at[pl.ds(off, SB)], mbb[sl], smb[sl])

        def wait(k, sl):
            off = pl.multiple_of(base + k * SB, SEG)
            pltpu.make_async_copy(dloc_hbm.at[pl.ds(off, SB)], dlb[sl].at[pl.ds(0, SB)], smb[sl]).wait()
            pltpu.make_async_copy(m_hbm.at[pl.ds(off, SB)], mbb[sl], smb[sl]).wait()

        @pl.when(nblk > 0)
        def _():
            issue(0, 0)

        @pl.when(nblk > 1)
        def _():
            issue(1, 1)

        def per_pair(g, c2):
            for sl in (0, 1):
                k = 2 * g + sl

                @pl.when(k < nblk)
                def _():
                    wait(k, sl)
                    nin = jnp.minimum(SB, seg_len - k * SB)

                    def per_e(e, c3):
                        r = _sgeti(dlb[sl], e)
                        for q in range(4):
                            cv = table[pl.ds(r * H + q * 16, 16)]
                            mv = mbb[sl][e, pl.ds(q * 16, 16)]
                            table[pl.ds(r * H + q * 16, 16)] = jnp.maximum(cv, mv)
                        return c3

                    lax.fori_loop(0, nin, per_e, 0)

                    @pl.when(k + 2 < nblk)
                    def _():
                        issue(k + 2, sl)

            return c2

        lax.fori_loop(0, (nblk + 1) >> 1, per_pair, 0)
        return carry

    lax.fori_loop(0, NW, per_worker, 0)

    @pl.when(b < NB - 1)
    def _():
        pltpu.sync_copy(table.at[pl.ds(0, NPB * H)], agg_hbm.at[pl.ds(b * NPB * H, NPB * H)])

    @pl.when(b == NB - 1)
    def _():
        pltpu.sync_copy(
            table.at[pl.ds(0, LASTROWS * H)],
            agg_hbm.at[pl.ds((NB - 1) * NPB * H, LASTROWS * H)],
        )


_sc_scatter = functools.partial(
    pl.kernel,
    mesh=plsc.VectorSubcoreMesh(core_axis_name="c", subcore_axis_name="s"),
    out_type=jax.ShapeDtypeStruct((N * H,), jnp.float32),
    scratch_types=[
        pltpu.VMEM((NPB * H,), jnp.float32),
        pltpu.VMEM((SB, H), jnp.float32),
        pltpu.VMEM((SB, H), jnp.float32),
        pltpu.VMEM((SB + 16,), jnp.int32),
        pltpu.VMEM((SB + 16,), jnp.int32),
        pltpu.VMEM((NW * NB + 16,), jnp.int32),
        pltpu.VMEM((NW * NB + 16,), jnp.int32),
        pltpu.SemaphoreType.DMA,
        pltpu.SemaphoreType.DMA,
    ],
    compiler_params=pltpu.CompilerParams(needs_layout_passes=False),
)(_scatter_body)


NBLK = 2000


def _proj_body(h_ref, w1_ref, w2_ref, t1_ref, t2_ref):
    h = h_ref[...]
    t1_ref[...] = jnp.dot(h, w1_ref[...], preferred_element_type=jnp.float32)
    t2_ref[...] = jnp.dot(h, w2_ref[...], preferred_element_type=jnp.float32)


def _node_tables(h, Wf, Ws):
    w1 = jnp.concatenate([Wf[0:H], Ws[0:H]], axis=1)
    w2 = jnp.concatenate([Wf[H : 2 * H], Ws[H : 2 * H]], axis=1)
    nb = pl.BlockSpec((NBLK, H), lambda i: (i, 0))
    tb = pl.BlockSpec((NBLK, 2 * H), lambda i: (i, 0))
    wb = pl.BlockSpec((H, 2 * H), lambda i: (0, 0))
    return pl.pallas_call(
        _proj_body,
        grid=(N // NBLK,),
        in_specs=[nb, wb, wb],
        out_specs=[tb, tb],
        out_shape=[
            jax.ShapeDtypeStruct((N, 2 * H), jnp.float32),
            jax.ShapeDtypeStruct((N, 2 * H), jnp.float32),
        ],
    )(h, w1, w2)


MBLK = 2048


def _msg_body(g1_ref, g2_ref, ea_ref, wfe_ref, wse_ref, bf_ref, bs_ref, m_ref):
    g1 = g1_ref[...]
    g2 = g2_ref[...]
    ea = ea_ref[:, 0:H]
    zf = (
        g1[:, 0:H]
        + g2[:, 0:H]
        + jnp.dot(ea, wfe_ref[...], preferred_element_type=jnp.float32)
        + bf_ref[...]
    )
    zs = (
        g1[:, H : 2 * H]
        + g2[:, H : 2 * H]
        + jnp.dot(ea, wse_ref[...], preferred_element_type=jnp.float32)
        + bs_ref[...]
    )
    softplus = jnp.maximum(zs, 0.0) + jnp.log1p(jnp.exp(-jnp.abs(zs)))
    m_ref[...] = jax.nn.sigmoid(zf) * softplus


def _messages(g1, g2, eap, Wf, bf, Ws, bs):
    gb = pl.BlockSpec((MBLK, 2 * H), lambda i: (i, 0))
    eb = pl.BlockSpec((MBLK, 2 * H), lambda i: (i, 0))
    mb = pl.BlockSpec((MBLK, H), lambda i: (i, 0))
    wb = pl.BlockSpec((H, H), lambda i: (0, 0))
    vb = pl.BlockSpec((1, H), lambda i: (0, 0))
    return pl.pallas_call(
        _msg_body,
        grid=(E_PAD2 // MBLK,),
        in_specs=[gb, gb, eb, wb, wb, vb, vb],
        out_specs=mb,
        out_shape=jax.ShapeDtypeStruct((E_PAD2, H), jnp.float32),
    )(g1, g2, eap, Wf[2 * H :], Ws[2 * H :], bf.reshape(1, H), bs.reshape(1, H))


def _cg_layer(h, dstg, srcg, dloc, pref, pcnt, eap, Wf, bf, Ws, bs, gamma, beta):
    t1, t2 = _node_tables(h, Wf, Ws)
    g1, g2 = _sc_gather(dstg, srcg, t1, t2)
    m = _messages(g1, g2, eap, Wf, bf, Ws, bs)
    agg = _sc_scatter(m, dloc, pref, pcnt).reshape(N, H)
    mu = agg.mean(axis=0)
    var = agg.var(axis=0)
    agg = (agg - mu) / jnp.sqrt(var + 1e-5) * gamma + beta
    return agg + h


def kernel(x, edge_index, edge_attr, Wn, bn_, We, be_, Wf1, bf1, Ws1, bs1, g1, b1, Wf2, bf2, Ws2, bs2, g2, b2, Wl, bl):
    src = edge_index[0]
    dst = edge_index[1]
    h = x @ Wn + bn_
    We2 = jnp.concatenate([We, jnp.zeros_like(We)], axis=1)
    be2 = jnp.concatenate([be_, jnp.zeros_like(be_)], axis=0)
    ea2 = edge_attr @ We2 + be2
    dstg, srcg, perm, dloc, pref, pcnt = _sc_bucketize(dst, src)
    eap = _sc_permute(perm, ea2)
    h = _cg_layer(h, dstg, srcg, dloc, pref, pcnt, eap, Wf1, bf1, Ws1, bs1, g1, b1)
    h = _cg_layer(h, dstg, srcg, dloc, pref, pcnt, eap, Wf2, bf2, Ws2, bs2, g2, b2)
    logits = h @ Wl + bl
    return (logits, h)


# speedup vs baseline: 1.1688x; 1.1688x over previous
import functools

import jax
import jax.numpy as jnp
from jax import lax
from jax.experimental import pallas as pl
from jax.experimental.pallas import tpu as pltpu
from jax.experimental.pallas import tpu_sc as plsc

N = 50000
E = 800000
H = 64

_info = plsc.get_sparse_core_info()
NC, NS = _info.num_cores, _info.num_subcores
NW = NC * NS
NB = NW
NPB = 1568
LASTROWS = N - (NB - 1) * NPB
EPW = E // NW
RW = 26112
E_PAD = NW * RW
E_PAD2 = E_PAD + 2048
BCH = 1000
GC = 256
SEG = 32


def _wid():
    return lax.axis_index("s") * NC + lax.axis_index("c")


def _bucket_of(d):
    return ((d >> 5) * 669) >> 15


_VECS = BCH // 16
_REM = BCH - _VECS * 16


def _lane_extract(vec_a, vec_b, b):
    return vec_a[b] if b < 16 else vec_b[b - 16]


def _bucketize_body(
    dst_hbm,
    src_hbm,
    dstg_hbm,
    srcg_hbm,
    perm_hbm,
    dloc_hbm,
    pref_hbm,
    pcnt_hbm,
    chd,
    chs,
    reg_d,
    reg_s,
    reg_p,
    reg_l,
    prefbuf,
    pcntbuf,
):
    w = _wid()
    wbase = w * EPW
    obase = w * RW
    iota = lax.iota(jnp.int32, 16)
    zero16 = jnp.zeros((16,), jnp.int32)

    def zr(k, carry):
        reg_d[pl.ds(k * 16, 16)] = zero16
        reg_s[pl.ds(k * 16, 16)] = zero16
        reg_p[pl.ds(k * 16, 16)] = zero16
        reg_l[pl.ds(k * 16, 16)] = zero16
        return carry

    lax.fori_loop(0, RW // 16, zr, 0)

    def pass_a(i, acc):
        acc_a, acc_b = acc
        pltpu.sync_copy(dst_hbm.at[pl.ds(wbase + i * BCH, BCH)], chd.at[pl.ds(0, BCH)])

        def inner(v, acc2):
            a2a, a2b = acc2
            d16 = chd[pl.ds(v * 16, 16)]
            nvalid = jnp.where(v == _VECS, _REM, 16)
            valid = iota < nvalid
            b16 = _bucket_of(d16)
            for b in range(NB):
                mk = jnp.logical_and(b16 == b, valid)
                pop = plsc.all_reduce_population_count(mk)
                oh = (iota == (b % 16)).astype(jnp.int32) * pop
                if b < 16:
                    a2a = a2a + oh
                else:
                    a2b = a2b + oh
            return (a2a, a2b)

        return lax.fori_loop(0, _VECS + 1, inner, (acc_a, acc_b))

    cnt_a, cnt_b = lax.fori_loop(0, EPW // BCH, pass_a, (zero16, zero16))

    pc_a = (cnt_a + (SEG - 1)) & (-SEG)
    pc_b = (cnt_b + (SEG - 1)) & (-SEG)
    inc_a = jnp.cumsum(pc_a)
    inc_b = jnp.cumsum(pc_b)
    ex_a = inc_a - pc_a
    ex_b = inc_b - pc_b + inc_a[15]

    prefbuf[pl.ds(0, 16)] = ex_a
    prefbuf[pl.ds(16, 16)] = ex_b
    pcntbuf[pl.ds(0, 16)] = pc_a
    pcntbuf[pl.ds(16, 16)] = pc_b
    pltpu.sync_copy(prefbuf, pref_hbm.at[pl.ds(w * NB, NB)])
    pltpu.sync_copy(pcntbuf, pcnt_hbm.at[pl.ds(w * NB, NB)])

    def pass_b(i, cur):
        cur_a, cur_b = cur
        cbase = wbase + i * BCH
        pltpu.sync_copy(dst_hbm.at[pl.ds(cbase, BCH)], chd.at[pl.ds(0, BCH)])
        pltpu.sync_copy(src_hbm.at[pl.ds(cbase, BCH)], chs.at[pl.ds(0, BCH)])

        def inner(v, cur2):
            c2a, c2b = cur2
            d16 = chd[pl.ds(v * 16, 16)]
            s16 = chs[pl.ds(v * 16, 16)]
            nvalid = jnp.where(v == _VECS, _REM, 16)
            valid = iota < nvalid
            b16 = _bucket_of(d16)
            pos = zero16
            for b in range(NB):
                mk = jnp.logical_and(b16 == b, valid)
                pc = jnp.cumsum(mk.astype(jnp.int32))
                pop = pc[15]
                cb = _lane_extract(c2a, c2b, b)
                pos = jnp.where(mk, cb + pc - 1, pos)
                oh = (iota == (b % 16)).astype(jnp.int32) * pop
                if b < 16:
                    c2a = c2a + oh
                else:
                    c2b = c2b + oh
            gid16 = cbase + v * 16 + iota
            dl16 = d16 - b16 * NPB
            plsc.store_scatter(reg_d, [pos], d16, mask=valid)
            plsc.store_scatter(reg_s, [pos], s16, mask=valid)
            plsc.store_scatter(reg_p, [pos], gid16, mask=valid)
            plsc.store_scatter(reg_l, [pos], dl16, mask=valid)
            return (c2a, c2b)

        return lax.fori_loop(0, _VECS + 1, inner, (cur_a, cur_b))

    lax.fori_loop(0, EPW // BCH, pass_b, (ex_a, ex_b))

    for b in range(NB):
        c = _lane_extract(cnt_a, cnt_b, b)
        pc = _lane_extract(pc_a, pc_b, b)
        start = _lane_extract(ex_a, ex_b, b)
        pad = pc - c
        seg_end = start + c

        @pl.when(pad > 0)
        def _():
            lastidx = jnp.full((16,), seg_end - 1, jnp.int32)
            lv_d = plsc.load_gather(reg_d, [lastidx])
            lv_s = plsc.load_gather(reg_s, [lastidx])
            lv_p = plsc.load_gather(reg_p, [lastidx])
            lv_l = plsc.load_gather(reg_l, [lastidx])
            for j in (0, 16):
                lanes = iota + j
                mask = lanes < pad
                idx = seg_end + lanes
                plsc.store_scatter(reg_d, [idx], lv_d, mask=mask)
                plsc.store_scatter(reg_s, [idx], lv_s, mask=mask)
                plsc.store_scatter(reg_p, [idx], lv_p, mask=mask)
                plsc.store_scatter(reg_l, [idx], lv_l, mask=mask)

    pltpu.sync_copy(reg_d, dstg_hbm.at[pl.ds(obase, RW)])
    pltpu.sync_copy(reg_s, srcg_hbm.at[pl.ds(obase, RW)])
    pltpu.sync_copy(reg_p, perm_hbm.at[pl.ds(obase, RW)])
    pltpu.sync_copy(reg_l, dloc_hbm.at[pl.ds(obase, RW)])


_sc_bucketize = functools.partial(
    pl.kernel,
    mesh=plsc.VectorSubcoreMesh(core_axis_name="c", subcore_axis_name="s"),
    out_type=[
        jax.ShapeDtypeStruct((E_PAD,), jnp.int32),
        jax.ShapeDtypeStruct((E_PAD,), jnp.int32),
        jax.ShapeDtypeStruct((E_PAD,), jnp.int32),
        jax.ShapeDtypeStruct((E_PAD2,), jnp.int32),
        jax.ShapeDtypeStruct((NW * NB,), jnp.int32),
        jax.ShapeDtypeStruct((NW * NB,), jnp.int32),
    ],
    scratch_types=[
        pltpu.VMEM((BCH + 16,), jnp.int32),
        pltpu.VMEM((BCH + 16,), jnp.int32),
        pltpu.VMEM((RW,), jnp.int32),
        pltpu.VMEM((RW,), jnp.int32),
        pltpu.VMEM((RW,), jnp.int32),
        pltpu.VMEM((RW,), jnp.int32),
        pltpu.VMEM((NB,), jnp.int32),
        pltpu.VMEM((NB,), jnp.int32),
    ],
    compiler_params=pltpu.CompilerParams(needs_layout_passes=False),
)(_bucketize_body)


def _permute_body(perm_hbm, ea2_hbm, eap_hbm, idx0, idx1, row0, row1, si0, si1, sg0, sg1, sw0, sw1):
    w = _wid()
    obase = w * RW
    nch = RW // GC
    idxb = (idx0, idx1)
    rowb = (row0, row1)
    sib = (si0, si1)
    sgb = (sg0, sg1)
    swb = (sw0, sw1)

    for b in (0, 1):
        pltpu.async_copy(perm_hbm.at[pl.ds(obase + b * GC, GC)], idxb[b], sib[b])
    pltpu.make_async_copy(perm_hbm.at[pl.ds(obase, GC)], idxb[0], sib[0]).wait()
    pltpu.async_copy(ea2_hbm.at[idxb[0]], rowb[0], sgb[0])

    def step(g, carry):
        for b in (1, 0):
            i = 2 * g + (1 if b == 1 else 2)
            pb = 1 - b

            @pl.when(i < nch)
            def _():
                base = obase + i * GC

                @pl.when(i >= 2)
                def _():
                    pltpu.make_async_copy(rowb[b], eap_hbm.at[pl.ds(base - 2 * GC, GC)], swb[b]).wait()

                pltpu.make_async_copy(perm_hbm.at[pl.ds(base, GC)], idxb[b], sib[b]).wait()
                pltpu.async_copy(ea2_hbm.at[idxb[b]], rowb[b], sgb[b])
                pltpu.make_async_copy(ea2_hbm.at[idxb[pb]], rowb[pb], sgb[pb]).wait()

                @pl.when(i + 1 < nch)
                def _():
                    pltpu.async_copy(perm_hbm.at[pl.ds(base + GC, GC)], idxb[pb], sib[pb])

                pltpu.async_copy(rowb[pb], eap_hbm.at[pl.ds(base - GC, GC)], swb[pb])
        return carry

    lax.fori_loop(0, nch // 2, step, 0)
    lastb = (nch - 1) & 1
    pltpu.make_async_copy(ea2_hbm.at[idxb[lastb]], rowb[lastb], sgb[lastb]).wait()
    pltpu.async_copy(rowb[lastb], eap_hbm.at[pl.ds(obase + (nch - 1) * GC, GC)], swb[lastb])
    for b in (0, 1):
        base = obase + (nch - 2 + b) * GC
        pltpu.make_async_copy(rowb[b], eap_hbm.at[pl.ds(base, GC)], swb[b]).wait()


_sc_permute = functools.partial(
    pl.kernel,
    mesh=plsc.VectorSubcoreMesh(core_axis_name="c", subcore_axis_name="s"),
    out_type=jax.ShapeDtypeStruct((E_PAD2, 2 * H), jnp.float32),
    scratch_types=[
        pltpu.VMEM((GC,), jnp.int32),
        pltpu.VMEM((GC,), jnp.int32),
        pltpu.VMEM((GC, 2 * H), jnp.float32),
        pltpu.VMEM((GC, 2 * H), jnp.float32),
        pltpu.SemaphoreType.DMA,
        pltpu.SemaphoreType.DMA,
        pltpu.SemaphoreType.DMA,
        pltpu.SemaphoreType.DMA,
        pltpu.SemaphoreType.DMA,
        pltpu.SemaphoreType.DMA,
    ],
)(_permute_body)


GGC = 192


def _gather_body(
    dstg_hbm, srcg_hbm, t1_hbm, t2_hbm, a_hbm,
    ixd0, ixd1, ixs0, ixs1, r10, r11, ab0, ab1, si0, si1, sg0, sg1, sw0, sw1,
):
    w = _wid()
    obase = w * RW
    nch = RW // GGC
    ixd = (ixd0, ixd1)
    ixs = (ixs0, ixs1)
    r1b = (r10, r11)
    abb = (ab0, ab1)
    sib = (si0, si1)
    sgb = (sg0, sg1)
    swb = (sw0, sw1)

    def issue_idx(i, b):
        base = obase + i * GGC
        pltpu.async_copy(dstg_hbm.at[pl.ds(base, GGC)], ixd[b], sib[b])
        pltpu.async_copy(srcg_hbm.at[pl.ds(base, GGC)], ixs[b], sib[b])

    def wait_idx(i, b):
        base = obase + i * GGC
        pltpu.make_async_copy(dstg_hbm.at[pl.ds(base, GGC)], ixd[b], sib[b]).wait()
        pltpu.make_async_copy(srcg_hbm.at[pl.ds(base, GGC)], ixs[b], sib[b]).wait()

    def issue_gather(b):
        pltpu.async_copy(t1_hbm.at[ixd[b]], r1b[b], sgb[b])
        pltpu.async_copy(t2_hbm.at[ixs[b]], abb[b], sgb[b])

    def wait_gather(b):
        pltpu.make_async_copy(t1_hbm.at[ixd[b]], r1b[b], sgb[b]).wait()
        pltpu.make_async_copy(t2_hbm.at[ixs[b]], abb[b], sgb[b]).wait()

    def compute_a(b):
        def row(r, carry):
            for q in range(8):
                abb[b][r, pl.ds(q * 16, 16)] = (
                    abb[b][r, pl.ds(q * 16, 16)] + r1b[b][r, pl.ds(q * 16, 16)]
                )
            return carry

        lax.fori_loop(0, GGC, row, 0)

    def issue_write(i, b):
        base = obase + i * GGC
        pltpu.async_copy(abb[b], a_hbm.at[pl.ds(base, GGC)], swb[b])

    def wait_write(i, b):
        base = obase + i * GGC
        pltpu.make_async_copy(abb[b], a_hbm.at[pl.ds(base, GGC)], swb[b]).wait()

    issue_idx(0, 0)
    issue_idx(1, 1)
    wait_idx(0, 0)
    issue_gather(0)

    def step(g, carry):
        for b in (1, 0):
            i = 2 * g + (1 if b == 1 else 2)
            pb = 1 - b

            @pl.when(i < nch)
            def _():
                @pl.when(i >= 2)
                def _():
                    wait_write(i - 2, b)

                wait_idx(i, b)
                issue_gather(b)
                wait_gather(pb)

                @pl.when(i + 1 < nch)
                def _():
                    issue_idx(i + 1, pb)

                compute_a(pb)
                issue_write(i - 1, pb)
        return carry

    lax.fori_loop(0, nch // 2, step, 0)
    lastb = (nch - 1) & 1
    wait_gather(lastb)
    compute_a(lastb)
    issue_write(nch - 1, lastb)
    wait_write(nch - 2, 1 - lastb)
    wait_write(nch - 1, lastb)


_sc_gather = functools.partial(
    pl.kernel,
    mesh=plsc.VectorSubcoreMesh(core_axis_name="c", subcore_axis_name="s"),
    out_type=jax.ShapeDtypeStruct((E_PAD2, 2 * H), jnp.float32),
    scratch_types=[
        pltpu.VMEM((GGC,), jnp.int32),
        pltpu.VMEM((GGC,), jnp.int32),
        pltpu.VMEM((GGC,), jnp.int32),
        pltpu.VMEM((GGC,), jnp.int32),
        pltpu.VMEM((GGC, 2 * H), jnp.float32),
        pltpu.VMEM((GGC, 2 * H), jnp.float32),
        pltpu.VMEM((GGC, 2 * H), jnp.float32),
        pltpu.VMEM((GGC, 2 * H), jnp.float32),
        pltpu.SemaphoreType.DMA,
        pltpu.SemaphoreType.DMA,
        pltpu.SemaphoreType.DMA,
        pltpu.SemaphoreType.DMA,
        pltpu.SemaphoreType.DMA,
        pltpu.SemaphoreType.DMA,
    ],
)(_gather_body)


def _sgeti(ref, i):
    return ref[pl.ds(i, 16)][0]


SB = 64


def _scatter_body(
    m_hbm, dloc_hbm, pref_hbm, pcnt_hbm, agg_hbm,
    table, mb0, mb1, dl0, dl1, prefv, pcntv, s0, s1,
):
    b = _wid()
    pltpu.sync_copy(pref_hbm, prefv.at[pl.ds(0, NW * NB)])
    pltpu.sync_copy(pcnt_hbm, pcntv.at[pl.ds(0, NW * NB)])
    mbb = (mb0, mb1)
    dlb = (dl0, dl1)
    smb = (s0, s1)

    zero16 = jnp.zeros((16,), jnp.float32)

    def zr(r, carry):
        table[pl.ds(r * 16, 16)] = zero16
        return carry

    lax.fori_loop(0, NPB * H // 16, zr, 0)

    def per_worker(u, carry):
        base = pl.multiple_of(u * RW + _sgeti(prefv, u * NB + b), SEG)
        seg_len = _sgeti(pcntv, u * NB + b)
        nblk = (seg_len + SB - 1) >> 6

        def issue(k, sl):
            off = pl.multiple_of(base + k * SB, SEG)
            pltpu.async_copy(dloc_hbm.at[pl.ds(off, SB)], dlb[sl].at[pl.ds(0, SB)], smb[sl])
            pltpu.async_copy(m_hbm.at[pl.ds(off, SB)], mbb[sl], smb[sl])

        def wait(k, sl):
            off = pl.multiple_of(base + k * SB, SEG)
            pltpu.make_async_copy(dloc_hbm.at[pl.ds(off, SB)], dlb[sl].at[pl.ds(0, SB)], smb[sl]).wait()
            pltpu.make_async_copy(m_hbm.at[pl.ds(off, SB)], mbb[sl], smb[sl]).wait()

        @pl.when(nblk > 0)
        def _():
            issue(0, 0)

        @pl.when(nblk > 1)
        def _():
            issue(1, 1)

        def per_pair(g, c2):
            for sl in (0, 1):
                k = 2 * g + sl

                @pl.when(k < nblk)
                def _():
                    wait(k, sl)
                    nin = jnp.minimum(SB, seg_len - k * SB)

                    def per_e(e, c3):
                        r = _sgeti(dlb[sl], e)
                        for q in range(4):
                            cv = table[pl.ds(r * H + q * 16, 16)]
                            mv = mbb[sl][e, pl.ds(q * 16, 16)]
                            table[pl.ds(r * H + q * 16, 16)] = jnp.maximum(cv, mv)
                        return c3

                    lax.fori_loop(0, nin, per_e, 0)

                    @pl.when(k + 2 < nblk)
                    def _():
                        issue(k + 2, sl)

            return c2

        lax.fori_loop(0, (nblk + 1) >> 1, per_pair, 0)
        return carry

    lax.fori_loop(0, NW, per_worker, 0)

    @pl.when(b < NB - 1)
    def _():
        pltpu.sync_copy(table.at[pl.ds(0, NPB * H)], agg_hbm.at[pl.ds(b * NPB * H, NPB * H)])

    @pl.when(b == NB - 1)
    def _():
        pltpu.sync_copy(
            table.at[pl.ds(0, LASTROWS * H)],
            agg_hbm.at[pl.ds((NB - 1) * NPB * H, LASTROWS * H)],
        )


_sc_scatter = functools.partial(
    pl.kernel,
    mesh=plsc.VectorSubcoreMesh(core_axis_name="c", subcore_axis_name="s"),
    out_type=jax.ShapeDtypeStruct((N * H,), jnp.float32),
    scratch_types=[
        pltpu.VMEM((NPB * H,), jnp.float32),
        pltpu.VMEM((SB, H), jnp.float32),
        pltpu.VMEM((SB, H), jnp.float32),
        pltpu.VMEM((SB + 16,), jnp.int32),
        pltpu.VMEM((SB + 16,), jnp.int32),
        pltpu.VMEM((NW * NB + 16,), jnp.int32),
        pltpu.VMEM((NW * NB + 16,), jnp.int32),
        pltpu.SemaphoreType.DMA,
        pltpu.SemaphoreType.DMA,
    ],
    compiler_params=pltpu.CompilerParams(needs_layout_passes=False),
)(_scatter_body)


NBLK = 2000


def _proj_body(h_ref, w1_ref, w2_ref, t1_ref, t2_ref):
    h = h_ref[...]
    t1_ref[...] = jnp.dot(h, w1_ref[...], preferred_element_type=jnp.float32)
    t2_ref[...] = jnp.dot(h, w2_ref[...], preferred_element_type=jnp.float32)


def _node_tables(h, Wf, Ws):
    w1 = jnp.concatenate([Wf[0:H], Ws[0:H]], axis=1)
    w2 = jnp.concatenate([Wf[H : 2 * H], Ws[H : 2 * H]], axis=1)
    nb = pl.BlockSpec((NBLK, H), lambda i: (i, 0))
    tb = pl.BlockSpec((NBLK, 2 * H), lambda i: (i, 0))
    wb = pl.BlockSpec((H, 2 * H), lambda i: (0, 0))
    return pl.pallas_call(
        _proj_body,
        grid=(N // NBLK,),
        in_specs=[nb, wb, wb],
        out_specs=[tb, tb],
        out_shape=[
            jax.ShapeDtypeStruct((N, 2 * H), jnp.float32),
            jax.ShapeDtypeStruct((N, 2 * H), jnp.float32),
        ],
    )(h, w1, w2)


MBLK = 2048


def _msg_body(a_ref, ea_ref, wfe_ref, wse_ref, bf_ref, bs_ref, m_ref):
    a = a_ref[...]
    ea = ea_ref[:, 0:H]
    zf = (
        a[:, 0:H]
        + jnp.dot(ea, wfe_ref[...], preferred_element_type=jnp.float32)
        + bf_ref[...]
    )
    zs = (
        a[:, H : 2 * H]
        + jnp.dot(ea, wse_ref[...], preferred_element_type=jnp.float32)
        + bs_ref[...]
    )
    softplus = jnp.maximum(zs, 0.0) + jnp.log1p(jnp.exp(-jnp.abs(zs)))
    m_ref[...] = jax.nn.sigmoid(zf) * softplus


def _messages(a, eap, Wf, bf, Ws, bs):
    gb = pl.BlockSpec((MBLK, 2 * H), lambda i: (i, 0))
    eb = pl.BlockSpec((MBLK, 2 * H), lambda i: (i, 0))
    mb = pl.BlockSpec((MBLK, H), lambda i: (i, 0))
    wb = pl.BlockSpec((H, H), lambda i: (0, 0))
    vb = pl.BlockSpec((1, H), lambda i: (0, 0))
    return pl.pallas_call(
        _msg_body,
        grid=(E_PAD2 // MBLK,),
        in_specs=[gb, eb, wb, wb, vb, vb],
        out_specs=mb,
        out_shape=jax.ShapeDtypeStruct((E_PAD2, H), jnp.float32),
    )(a, eap, Wf[2 * H :], Ws[2 * H :], bf.reshape(1, H), bs.reshape(1, H))


def _cg_layer(h, dstg, srcg, dloc, pref, pcnt, eap, Wf, bf, Ws, bs, gamma, beta):
    t1, t2 = _node_tables(h, Wf, Ws)
    a = _sc_gather(dstg, srcg, t1, t2)
    m = _messages(a, eap, Wf, bf, Ws, bs)
    agg = _sc_scatter(m, dloc, pref, pcnt).reshape(N, H)
    mu = agg.mean(axis=0)
    var = agg.var(axis=0)
    agg = (agg - mu) / jnp.sqrt(var + 1e-5) * gamma + beta
    return agg + h


def kernel(x, edge_index, edge_attr, Wn, bn_, We, be_, Wf1, bf1, Ws1, bs1, g1, b1, Wf2, bf2, Ws2, bs2, g2, b2, Wl, bl):
    src = edge_index[0]
    dst = edge_index[1]
    h = x @ Wn + bn_
    We2 = jnp.concatenate([We, jnp.zeros_like(We)], axis=1)
    be2 = jnp.concatenate([be_, jnp.zeros_like(be_)], axis=0)
    ea2 = edge_attr @ We2 + be2
    dstg, srcg, perm, dloc, pref, pcnt = _sc_bucketize(dst, src)
    eap = _sc_permute(perm, ea2)
    h = _cg_layer(h, dstg, srcg, dloc, pref, pcnt, eap, Wf1, bf1, Ws1, bs1, g1, b1)
    h = _cg_layer(h, dstg, srcg, dloc, pref, pcnt, eap, Wf2, bf2, Ws2, bs2, g2, b2)
    logits = h @ Wl + bl
    return (logits, h)


# speedup vs baseline: 1.2810x; 1.0961x over previous
import functools

import jax
import jax.numpy as jnp
from jax import lax
from jax.experimental import pallas as pl
from jax.experimental.pallas import tpu as pltpu
from jax.experimental.pallas import tpu_sc as plsc

N = 50000
E = 800000
H = 64

_info = plsc.get_sparse_core_info()
NC, NS = _info.num_cores, _info.num_subcores
NW = NC * NS
NB = NW
NPB = 1568
LASTROWS = N - (NB - 1) * NPB
EPW = E // NW
RW = 26112
E_PAD = NW * RW
E_PAD2 = E_PAD + 2048
BCH = 1000
GC = 256
SEG = 32


def _wid():
    return lax.axis_index("s") * NC + lax.axis_index("c")


def _bucket_of(d):
    return ((d >> 5) * 669) >> 15


_VECS = BCH // 16
_REM = BCH - _VECS * 16


def _lane_extract(vec_a, vec_b, b):
    return vec_a[b] if b < 16 else vec_b[b - 16]


def _bucketize_body(
    dst_hbm,
    src_hbm,
    dstg_hbm,
    srcg_hbm,
    perm_hbm,
    dloc_hbm,
    pref_hbm,
    pcnt_hbm,
    chd,
    chs,
    reg_d,
    reg_s,
    reg_p,
    reg_l,
    prefbuf,
    pcntbuf,
):
    w = _wid()
    wbase = w * EPW
    obase = w * RW
    iota = lax.iota(jnp.int32, 16)
    zero16 = jnp.zeros((16,), jnp.int32)

    def zr(k, carry):
        reg_d[pl.ds(k * 16, 16)] = zero16
        reg_s[pl.ds(k * 16, 16)] = zero16
        reg_p[pl.ds(k * 16, 16)] = zero16
        reg_l[pl.ds(k * 16, 16)] = zero16
        return carry

    lax.fori_loop(0, RW // 16, zr, 0)

    def pass_a(i, acc):
        acc_a, acc_b = acc
        pltpu.sync_copy(dst_hbm.at[pl.ds(wbase + i * BCH, BCH)], chd.at[pl.ds(0, BCH)])

        def inner(v, acc2):
            a2a, a2b = acc2
            d16 = chd[pl.ds(v * 16, 16)]
            nvalid = jnp.where(v == _VECS, _REM, 16)
            valid = iota < nvalid
            b16 = _bucket_of(d16)
            for b in range(NB):
                mk = jnp.logical_and(b16 == b, valid)
                pop = plsc.all_reduce_population_count(mk)
                oh = (iota == (b % 16)).astype(jnp.int32) * pop
                if b < 16:
                    a2a = a2a + oh
                else:
                    a2b = a2b + oh
            return (a2a, a2b)

        return lax.fori_loop(0, _VECS + 1, inner, (acc_a, acc_b))

    cnt_a, cnt_b = lax.fori_loop(0, EPW // BCH, pass_a, (zero16, zero16))

    pc_a = (cnt_a + (SEG - 1)) & (-SEG)
    pc_b = (cnt_b + (SEG - 1)) & (-SEG)
    inc_a = jnp.cumsum(pc_a)
    inc_b = jnp.cumsum(pc_b)
    ex_a = inc_a - pc_a
    ex_b = inc_b - pc_b + inc_a[15]

    prefbuf[pl.ds(0, 16)] = ex_a
    prefbuf[pl.ds(16, 16)] = ex_b
    pcntbuf[pl.ds(0, 16)] = pc_a
    pcntbuf[pl.ds(16, 16)] = pc_b
    pltpu.sync_copy(prefbuf, pref_hbm.at[pl.ds(w * NB, NB)])
    pltpu.sync_copy(pcntbuf, pcnt_hbm.at[pl.ds(w * NB, NB)])

    def pass_b(i, cur):
        cur_a, cur_b = cur
        cbase = wbase + i * BCH
        pltpu.sync_copy(dst_hbm.at[pl.ds(cbase, BCH)], chd.at[pl.ds(0, BCH)])
        pltpu.sync_copy(src_hbm.at[pl.ds(cbase, BCH)], chs.at[pl.ds(0, BCH)])

        def inner(v, cur2):
            c2a, c2b = cur2
            d16 = chd[pl.ds(v * 16, 16)]
            s16 = chs[pl.ds(v * 16, 16)]
            nvalid = jnp.where(v == _VECS, _REM, 16)
            valid = iota < nvalid
            b16 = _bucket_of(d16)
            pos = zero16
            for b in range(NB):
                mk = jnp.logical_and(b16 == b, valid)
                pc = jnp.cumsum(mk.astype(jnp.int32))
                pop = pc[15]
                cb = _lane_extract(c2a, c2b, b)
                pos = jnp.where(mk, cb + pc - 1, pos)
                oh = (iota == (b % 16)).astype(jnp.int32) * pop
                if b < 16:
                    c2a = c2a + oh
                else:
                    c2b = c2b + oh
            gid16 = cbase + v * 16 + iota
            dl16 = d16 - b16 * NPB
            plsc.store_scatter(reg_d, [pos], d16, mask=valid)
            plsc.store_scatter(reg_s, [pos], s16, mask=valid)
            plsc.store_scatter(reg_p, [pos], gid16, mask=valid)
            plsc.store_scatter(reg_l, [pos], dl16, mask=valid)
            return (c2a, c2b)

        return lax.fori_loop(0, _VECS + 1, inner, (cur_a, cur_b))

    lax.fori_loop(0, EPW // BCH, pass_b, (ex_a, ex_b))

    for b in range(NB):
        c = _lane_extract(cnt_a, cnt_b, b)
        pc = _lane_extract(pc_a, pc_b, b)
        start = _lane_extract(ex_a, ex_b, b)
        pad = pc - c
        seg_end = start + c

        @pl.when(pad > 0)
        def _():
            lastidx = jnp.full((16,), seg_end - 1, jnp.int32)
            lv_d = plsc.load_gather(reg_d, [lastidx])
            lv_s = plsc.load_gather(reg_s, [lastidx])
            lv_p = plsc.load_gather(reg_p, [lastidx])
            lv_l = plsc.load_gather(reg_l, [lastidx])
            for j in (0, 16):
                lanes = iota + j
                mask = lanes < pad
                idx = seg_end + lanes
                plsc.store_scatter(reg_d, [idx], lv_d, mask=mask)
                plsc.store_scatter(reg_s, [idx], lv_s, mask=mask)
                plsc.store_scatter(reg_p, [idx], lv_p, mask=mask)
                plsc.store_scatter(reg_l, [idx], lv_l, mask=mask)

    pltpu.sync_copy(reg_d, dstg_hbm.at[pl.ds(obase, RW)])
    pltpu.sync_copy(reg_s, srcg_hbm.at[pl.ds(obase, RW)])
    pltpu.sync_copy(reg_p, perm_hbm.at[pl.ds(obase, RW)])
    pltpu.sync_copy(reg_l, dloc_hbm.at[pl.ds(obase, RW)])


_sc_bucketize = functools.partial(
    pl.kernel,
    mesh=plsc.VectorSubcoreMesh(core_axis_name="c", subcore_axis_name="s"),
    out_type=[
        jax.ShapeDtypeStruct((E_PAD,), jnp.int32),
        jax.ShapeDtypeStruct((E_PAD,), jnp.int32),
        jax.ShapeDtypeStruct((E_PAD,), jnp.int32),
        jax.ShapeDtypeStruct((E_PAD2,), jnp.int32),
        jax.ShapeDtypeStruct((NW * NB,), jnp.int32),
        jax.ShapeDtypeStruct((NW * NB,), jnp.int32),
    ],
    scratch_types=[
        pltpu.VMEM((BCH + 16,), jnp.int32),
        pltpu.VMEM((BCH + 16,), jnp.int32),
        pltpu.VMEM((RW,), jnp.int32),
        pltpu.VMEM((RW,), jnp.int32),
        pltpu.VMEM((RW,), jnp.int32),
        pltpu.VMEM((RW,), jnp.int32),
        pltpu.VMEM((NB,), jnp.int32),
        pltpu.VMEM((NB,), jnp.int32),
    ],
    compiler_params=pltpu.CompilerParams(needs_layout_passes=False),
)(_bucketize_body)


def _permute_body(perm_hbm, ea2_hbm, eap_hbm, idx0, idx1, row0, row1, si0, si1, sg0, sg1, sw0, sw1):
    w = _wid()
    obase = w * RW
    nch = RW // GC
    idxb = (idx0, idx1)
    rowb = (row0, row1)
    sib = (si0, si1)
    sgb = (sg0, sg1)
    swb = (sw0, sw1)

    for b in (0, 1):
        pltpu.async_copy(perm_hbm.at[pl.ds(obase + b * GC, GC)], idxb[b], sib[b])
    pltpu.make_async_copy(perm_hbm.at[pl.ds(obase, GC)], idxb[0], sib[0]).wait()
    pltpu.async_copy(ea2_hbm.at[idxb[0]], rowb[0], sgb[0])

    def step(g, carry):
        for b in (1, 0):
            i = 2 * g + (1 if b == 1 else 2)
            pb = 1 - b

            @pl.when(i < nch)
            def _():
                base = obase + i * GC

                @pl.when(i >= 2)
                def _():
                    pltpu.make_async_copy(rowb[b], eap_hbm.at[pl.ds(base - 2 * GC, GC)], swb[b]).wait()

                pltpu.make_async_copy(perm_hbm.at[pl.ds(base, GC)], idxb[b], sib[b]).wait()
                pltpu.async_copy(ea2_hbm.at[idxb[b]], rowb[b], sgb[b])
                pltpu.make_async_copy(ea2_hbm.at[idxb[pb]], rowb[pb], sgb[pb]).wait()

                @pl.when(i + 1 < nch)
                def _():
                    pltpu.async_copy(perm_hbm.at[pl.ds(base + GC, GC)], idxb[pb], sib[pb])

                pltpu.async_copy(rowb[pb], eap_hbm.at[pl.ds(base - GC, GC)], swb[pb])
        return carry

    lax.fori_loop(0, nch // 2, step, 0)
    lastb = (nch - 1) & 1
    pltpu.make_async_copy(ea2_hbm.at[idxb[lastb]], rowb[lastb], sgb[lastb]).wait()
    pltpu.async_copy(rowb[lastb], eap_hbm.at[pl.ds(obase + (nch - 1) * GC, GC)], swb[lastb])
    for b in (0, 1):
        base = obase + (nch - 2 + b) * GC
        pltpu.make_async_copy(rowb[b], eap_hbm.at[pl.ds(base, GC)], swb[b]).wait()


_sc_permute = functools.partial(
    pl.kernel,
    mesh=plsc.VectorSubcoreMesh(core_axis_name="c", subcore_axis_name="s"),
    out_type=jax.ShapeDtypeStruct((E_PAD2, 2 * H), jnp.float32),
    scratch_types=[
        pltpu.VMEM((GC,), jnp.int32),
        pltpu.VMEM((GC,), jnp.int32),
        pltpu.VMEM((GC, 2 * H), jnp.float32),
        pltpu.VMEM((GC, 2 * H), jnp.float32),
        pltpu.SemaphoreType.DMA,
        pltpu.SemaphoreType.DMA,
        pltpu.SemaphoreType.DMA,
        pltpu.SemaphoreType.DMA,
        pltpu.SemaphoreType.DMA,
        pltpu.SemaphoreType.DMA,
    ],
)(_permute_body)


GGC = 192


def _gather_body(
    dstg_hbm, srcg_hbm, t1_hbm, t2_hbm, a_hbm,
    ixd0, ixd1, ixs0, ixs1, r10, r11, ab0, ab1, si0, si1, sg0, sg1, sw0, sw1,
):
    w = _wid()
    obase = w * RW
    nch = RW // GGC
    ixd = (ixd0, ixd1)
    ixs = (ixs0, ixs1)
    r1b = (r10, r11)
    abb = (ab0, ab1)
    sib = (si0, si1)
    sgb = (sg0, sg1)
    swb = (sw0, sw1)

    def issue_idx(i, b):
        base = obase + i * GGC
        pltpu.async_copy(dstg_hbm.at[pl.ds(base, GGC)], ixd[b], sib[b])
        pltpu.async_copy(srcg_hbm.at[pl.ds(base, GGC)], ixs[b], sib[b])

    def wait_idx(i, b):
        base = obase + i * GGC
        pltpu.make_async_copy(dstg_hbm.at[pl.ds(base, GGC)], ixd[b], sib[b]).wait()
        pltpu.make_async_copy(srcg_hbm.at[pl.ds(base, GGC)], ixs[b], sib[b]).wait()

    def issue_gather(b):
        pltpu.async_copy(t1_hbm.at[ixd[b]], r1b[b], sgb[b])
        pltpu.async_copy(t2_hbm.at[ixs[b]], abb[b], sgb[b])

    def wait_gather(b):
        pltpu.make_async_copy(t1_hbm.at[ixd[b]], r1b[b], sgb[b]).wait()
        pltpu.make_async_copy(t2_hbm.at[ixs[b]], abb[b], sgb[b]).wait()

    def compute_a(b):
        def row(r, carry):
            for q in range(8):
                abb[b][r, pl.ds(q * 16, 16)] = (
                    abb[b][r, pl.ds(q * 16, 16)] + r1b[b][r, pl.ds(q * 16, 16)]
                )
            return carry

        lax.fori_loop(0, GGC, row, 0)

    def issue_write(i, b):
        base = obase + i * GGC
        pltpu.async_copy(abb[b], a_hbm.at[pl.ds(base, GGC)], swb[b])

    def wait_write(i, b):
        base = obase + i * GGC
        pltpu.make_async_copy(abb[b], a_hbm.at[pl.ds(base, GGC)], swb[b]).wait()

    issue_idx(0, 0)
    issue_idx(1, 1)
    wait_idx(0, 0)
    issue_gather(0)

    def step(g, carry):
        for b in (1, 0):
            i = 2 * g + (1 if b == 1 else 2)
            pb = 1 - b

            @pl.when(i < nch)
            def _():
                @pl.when(i >= 2)
                def _():
                    wait_write(i - 2, b)

                wait_idx(i, b)
                issue_gather(b)
                wait_gather(pb)

                @pl.when(i + 1 < nch)
                def _():
                    issue_idx(i + 1, pb)

                compute_a(pb)
                issue_write(i - 1, pb)
        return carry

    lax.fori_loop(0, nch // 2, step, 0)
    lastb = (nch - 1) & 1
    wait_gather(lastb)
    compute_a(lastb)
    issue_write(nch - 1, lastb)
    wait_write(nch - 2, 1 - lastb)
    wait_write(nch - 1, lastb)


_sc_gather = functools.partial(
    pl.kernel,
    mesh=plsc.VectorSubcoreMesh(core_axis_name="c", subcore_axis_name="s"),
    out_type=jax.ShapeDtypeStruct((E_PAD2, 2 * H), jnp.float32),
    scratch_types=[
        pltpu.VMEM((GGC,), jnp.int32),
        pltpu.VMEM((GGC,), jnp.int32),
        pltpu.VMEM((GGC,), jnp.int32),
        pltpu.VMEM((GGC,), jnp.int32),
        pltpu.VMEM((GGC, 2 * H), jnp.float32),
        pltpu.VMEM((GGC, 2 * H), jnp.float32),
        pltpu.VMEM((GGC, 2 * H), jnp.float32),
        pltpu.VMEM((GGC, 2 * H), jnp.float32),
        pltpu.SemaphoreType.DMA,
        pltpu.SemaphoreType.DMA,
        pltpu.SemaphoreType.DMA,
        pltpu.SemaphoreType.DMA,
        pltpu.SemaphoreType.DMA,
        pltpu.SemaphoreType.DMA,
    ],
)(_gather_body)


def _sgeti(ref, i):
    return ref[pl.ds(i, 16)][0]


SB = 64


def _scatter_body(
    m_hbm, dloc_hbm, pref_hbm, pcnt_hbm, agg_hbm,
    table, mb0, mb1, dl0, dl1, prefv, pcntv, s0, s1,
):
    b = _wid()
    pltpu.sync_copy(pref_hbm, prefv.at[pl.ds(0, NW * NB)])
    pltpu.sync_copy(pcnt_hbm, pcntv.at[pl.ds(0, NW * NB)])
    mbb = (mb0, mb1)
    dlb = (dl0, dl1)
    smb = (s0, s1)

    zero16 = jnp.zeros((16,), jnp.float32)

    def zr(r, carry):
        table[pl.ds(r * 16, 16)] = zero16
        return carry

    lax.fori_loop(0, NPB * H // 16, zr, 0)

    def per_worker(u, carry):
        base = pl.multiple_of(u * RW + _sgeti(prefv, u * NB + b), SEG)
        seg_len = _sgeti(pcntv, u * NB + b)
        nblk = (seg_len + SB - 1) >> 6

        def issue(k, sl):
            off = pl.multiple_of(base + k * SB, SEG)
            pltpu.async_copy(dloc_hbm.at[pl.ds(off, SB)], dlb[sl].at[pl.ds(0, SB)], smb[sl])
            pltpu.async_copy(m_hbm.at[pl.ds(off, SB)], mbb[sl], smb[sl])

        def wait(k, sl):
            off = pl.multiple_of(base + k * SB, SEG)
            pltpu.make_async_copy(dloc_hbm.at[pl.ds(off, SB)], dlb[sl].at[pl.ds(0, SB)], smb[sl]).wait()
            pltpu.make_async_copy(m_hbm.at[pl.ds(off, SB)], mbb[sl], smb[sl]).wait()

        @pl.when(nblk > 0)
        def _():
            issue(0, 0)

        @pl.when(nblk > 1)
        def _():
            issue(1, 1)

        def per_pair(g, c2):
            for sl in (0, 1):
                k = 2 * g + sl

                @pl.when(k < nblk)
                def _():
                    wait(k, sl)
                    nin = jnp.minimum(SB, seg_len - k * SB)

                    def per_g(gidx, c3):
                        dl16 = dlb[sl][pl.ds(gidx * 16, 16)]
                        for lane in range(16):
                            r = dl16[lane]
                            e = gidx * 16 + lane
                            for q in range(4):
                                cv = table[pl.ds(r * H + q * 16, 16)]
                                mv = mbb[sl][e, pl.ds(q * 16, 16)]
                                table[pl.ds(r * H + q * 16, 16)] = jnp.maximum(cv, mv)
                        return c3

                    lax.fori_loop(0, nin >> 4, per_g, 0)

                    @pl.when(k + 2 < nblk)
                    def _():
                        issue(k + 2, sl)

            return c2

        lax.fori_loop(0, (nblk + 1) >> 1, per_pair, 0)
        return carry

    lax.fori_loop(0, NW, per_worker, 0)

    @pl.when(b < NB - 1)
    def _():
        pltpu.sync_copy(table.at[pl.ds(0, NPB * H)], agg_hbm.at[pl.ds(b * NPB * H, NPB * H)])

    @pl.when(b == NB - 1)
    def _():
        pltpu.sync_copy(
            table.at[pl.ds(0, LASTROWS * H)],
            agg_hbm.at[pl.ds((NB - 1) * NPB * H, LASTROWS * H)],
        )


_sc_scatter = functools.partial(
    pl.kernel,
    mesh=plsc.VectorSubcoreMesh(core_axis_name="c", subcore_axis_name="s"),
    out_type=jax.ShapeDtypeStruct((N * H,), jnp.float32),
    scratch_types=[
        pltpu.VMEM((NPB * H,), jnp.float32),
        pltpu.VMEM((SB, H), jnp.float32),
        pltpu.VMEM((SB, H), jnp.float32),
        pltpu.VMEM((SB + 16,), jnp.int32),
        pltpu.VMEM((SB + 16,), jnp.int32),
        pltpu.VMEM((NW * NB + 16,), jnp.int32),
        pltpu.VMEM((NW * NB + 16,), jnp.int32),
        pltpu.SemaphoreType.DMA,
        pltpu.SemaphoreType.DMA,
    ],
    compiler_params=pltpu.CompilerParams(needs_layout_passes=False),
)(_scatter_body)


NBLK = 2000


def _proj_body(h_ref, w1_ref, w2_ref, t1_ref, t2_ref):
    h = h_ref[...]
    t1_ref[...] = jnp.dot(h, w1_ref[...], preferred_element_type=jnp.float32)
    t2_ref[...] = jnp.dot(h, w2_ref[...], preferred_element_type=jnp.float32)


def _node_tables(h, Wf, Ws):
    w1 = jnp.concatenate([Wf[0:H], Ws[0:H]], axis=1)
    w2 = jnp.concatenate([Wf[H : 2 * H], Ws[H : 2 * H]], axis=1)
    nb = pl.BlockSpec((NBLK, H), lambda i: (i, 0))
    tb = pl.BlockSpec((NBLK, 2 * H), lambda i: (i, 0))
    wb = pl.BlockSpec((H, 2 * H), lambda i: (0, 0))
    return pl.pallas_call(
        _proj_body,
        grid=(N // NBLK,),
        in_specs=[nb, wb, wb],
        out_specs=[tb, tb],
        out_shape=[
            jax.ShapeDtypeStruct((N, 2 * H), jnp.float32),
            jax.ShapeDtypeStruct((N, 2 * H), jnp.float32),
        ],
    )(h, w1, w2)


MBLK = 2048


def _msg_body(a_ref, ea_ref, wfe_ref, wse_ref, bf_ref, bs_ref, m_ref):
    a = a_ref[...]
    ea = ea_ref[:, 0:H]
    zf = (
        a[:, 0:H]
        + jnp.dot(ea, wfe_ref[...], preferred_element_type=jnp.float32)
        + bf_ref[...]
    )
    zs = (
        a[:, H : 2 * H]
        + jnp.dot(ea, wse_ref[...], preferred_element_type=jnp.float32)
        + bs_ref[...]
    )
    softplus = jnp.maximum(zs, 0.0) + jnp.log1p(jnp.exp(-jnp.abs(zs)))
    m_ref[...] = jax.nn.sigmoid(zf) * softplus


def _messages(a, eap, Wf, bf, Ws, bs):
    gb = pl.BlockSpec((MBLK, 2 * H), lambda i: (i, 0))
    eb = pl.BlockSpec((MBLK, 2 * H), lambda i: (i, 0))
    mb = pl.BlockSpec((MBLK, H), lambda i: (i, 0))
    wb = pl.BlockSpec((H, H), lambda i: (0, 0))
    vb = pl.BlockSpec((1, H), lambda i: (0, 0))
    return pl.pallas_call(
        _msg_body,
        grid=(E_PAD2 // MBLK,),
        in_specs=[gb, eb, wb, wb, vb, vb],
        out_specs=mb,
        out_shape=jax.ShapeDtypeStruct((E_PAD2, H), jnp.float32),
    )(a, eap, Wf[2 * H :], Ws[2 * H :], bf.reshape(1, H), bs.reshape(1, H))


def _cg_layer(h, dstg, srcg, dloc, pref, pcnt, eap, Wf, bf, Ws, bs, gamma, beta):
    t1, t2 = _node_tables(h, Wf, Ws)
    a = _sc_gather(dstg, srcg, t1, t2)
    m = _messages(a, eap, Wf, bf, Ws, bs)
    agg = _sc_scatter(m, dloc, pref, pcnt).reshape(N, H)
    mu = agg.mean(axis=0)
    var = agg.var(axis=0)
    agg = (agg - mu) / jnp.sqrt(var + 1e-5) * gamma + beta
    return agg + h


def kernel(x, edge_index, edge_attr, Wn, bn_, We, be_, Wf1, bf1, Ws1, bs1, g1, b1, Wf2, bf2, Ws2, bs2, g2, b2, Wl, bl):
    src = edge_index[0]
    dst = edge_index[1]
    h = x @ Wn + bn_
    We2 = jnp.concatenate([We, jnp.zeros_like(We)], axis=1)
    be2 = jnp.concatenate([be_, jnp.zeros_like(be_)], axis=0)
    ea2 = edge_attr @ We2 + be2
    dstg, srcg, perm, dloc, pref, pcnt = _sc_bucketize(dst, src)
    eap = _sc_permute(perm, ea2)
    h = _cg_layer(h, dstg, srcg, dloc, pref, pcnt, eap, Wf1, bf1, Ws1, bs1, g1, b1)
    h = _cg_layer(h, dstg, srcg, dloc, pref, pcnt, eap, Wf2, bf2, Ws2, bs2, g2, b2)
    logits = h @ Wl + bl
    return (logits, h)
